# baseline (speedup 1.0000x reference)
import functools

import jax
import jax.numpy as jnp
import numpy as np
from jax import lax
from jax.experimental import pallas as pl
from jax.experimental.pallas import tpu as pltpu
from jax.experimental.pallas import tpu_sc as plsc

F32 = jnp.float32


def _fold(layer):
    s = layer['gamma'] / jnp.sqrt(layer['var'] + 1e-5)
    w = layer['W'] * s[:, None]
    b = (layer['b'] - layer['mean']) * s + layer['beta']
    return w.T, b


def _fps_body(xyz_ref, out_ref, *, npoint):
    xyz = xyz_ref[...]
    _, B, N = xyz.shape
    lane_n = jax.lax.broadcasted_iota(jnp.int32, (B, N), 1)
    lane_s = jax.lax.broadcasted_iota(jnp.int32, (3, B, npoint), 2)

    def body(i, st):
        dist, far, acc = st
        onehot = (lane_n == far).astype(F32)
        c = jnp.sum(xyz * onehot[None, :, :], axis=2, keepdims=True)
        acc = jnp.where(lane_s == i, c, acc)
        diff = xyz - c
        d = jnp.sum(diff * diff, axis=0)
        dist = jnp.minimum(dist, d)
        m = jnp.max(dist, axis=1, keepdims=True)
        far = jnp.min(jnp.where(dist == m, lane_n, N), axis=1, keepdims=True)
        return dist, far, acc

    dist0 = jnp.full((B, N), 1e10, F32)
    far0 = jnp.zeros((B, 1), jnp.int32)
    acc0 = jnp.zeros((3, B, npoint), F32)
    _, _, acc = jax.lax.fori_loop(0, npoint, body, (dist0, far0, acc0))
    out_ref[...] = acc


def _fps(xyz3bn, npoint):
    _, B, N = xyz3bn.shape
    return pl.pallas_call(
        functools.partial(_fps_body, npoint=npoint),
        out_shape=jax.ShapeDtypeStruct((3, B, npoint), F32),
    )(xyz3bn)


def _ball_body(naug_ref, xyz3_ref, feats_ref, lt_ref, *rest, r2s, Ks, N, S,
               emit_g):
    nscale = len(Ks)
    if emit_g:
        w1s = rest[:nscale]
        wbs = rest[nscale:2 * nscale]
        idx_refs = rest[2 * nscale:3 * nscale]
        tgt_refs = rest[3 * nscale:4 * nscale]
        g_refs = rest[4 * nscale:5 * nscale]
        bias_refs = rest[5 * nscale:6 * nscale]
    else:
        w1s = g_refs = None
        wbs = rest[:nscale]
        idx_refs = rest[nscale:2 * nscale]
        tgt_refs = rest[2 * nscale:3 * nscale]
        bias_refs = rest[3 * nscale:4 * nscale]

    b = pl.program_id(0)
    naug = naug_ref[0]
    x3 = xyz3_ref[0]
    lt = lt_ref[...]

    nxyz = naug[:, :3]
    sc = jnp.sum(nxyz * nxyz, axis=1, keepdims=True)
    sx = jnp.sum(x3 * x3, axis=0, keepdims=True)
    cross = jnp.dot(nxyz, x3, preferred_element_type=F32)
    sq = (sc + sx) - 2.0 * cross

    lane_j = jax.lax.broadcasted_iota(jnp.int32, (S, N), 1)
    row_s = jax.lax.broadcasted_iota(jnp.int32, (S, N), 0)
    M = pl.num_programs(0) * S
    for i in range(nscale):
        K = Ks[i]
        validb = sq <= r2s[i]
        valid = validb.astype(F32)
        rc = jnp.dot(valid, lt, preferred_element_type=F32)
        inr = jnp.logical_and(validb, rc <= float(K))
        tgt = jnp.where(inr,
                        (b * S + row_s) * K + (rc.astype(jnp.int32) - 1),
                        M * K)
        tgt_refs[i][0] = tgt
        first = jnp.min(jnp.where(validb, lane_j, N), axis=1, keepdims=True)
        idx_refs[i][0] = jnp.broadcast_to(first + b * N, (S, K))
        if emit_g:
            g_refs[i][0] = jnp.dot(feats_ref[0], w1s[i][...],
                                   preferred_element_type=F32)
        bias_refs[i][0] = jnp.dot(naug, wbs[i][...],
                                  preferred_element_type=F32)


def _ball(naug, xyz3, feats, lt, w1s, wbs, o1s, r2s, Ks, emit_g):
    B, S, _ = naug.shape
    N = xyz3.shape[2]
    Cp = feats.shape[2]
    nscale = len(Ks)
    c0 = lambda i: (i, 0, 0)
    k2 = lambda i: (0, 0)
    in_specs = [
        pl.BlockSpec((1, S, 8), c0),
        pl.BlockSpec((1, 3, N), c0),
        pl.BlockSpec((1, N, Cp), c0),
        pl.BlockSpec((N, N), k2),
    ]
    if emit_g:
        in_specs += [pl.BlockSpec((Cp, o), k2) for o in o1s]
    in_specs += [pl.BlockSpec((8, o), k2) for o in o1s]
    out_specs = [pl.BlockSpec((1, S, Ks[i]), c0) for i in range(nscale)]
    out_shape = [jax.ShapeDtypeStruct((B, S, Ks[i]), jnp.int32)
                 for i in range(nscale)]
    out_specs += [pl.BlockSpec((1, S, N), c0) for _ in range(nscale)]
    out_shape += [jax.ShapeDtypeStruct((B, S, N), jnp.int32)
                  for _ in range(nscale)]
    if emit_g:
        out_specs += [pl.BlockSpec((1, N, o), c0) for o in o1s]
        out_shape += [jax.ShapeDtypeStruct((B, N, o), F32) for o in o1s]
    out_specs += [pl.BlockSpec((1, S, o), c0) for o in o1s]
    out_shape += [jax.ShapeDtypeStruct((B, S, o), F32) for o in o1s]
    args = [naug, xyz3, feats, lt] + (list(w1s) if emit_g else []) + list(wbs)
    return pl.pallas_call(
        functools.partial(_ball_body, r2s=tuple(r2s), Ks=tuple(Ks), N=N, S=S,
                          emit_g=emit_g),
        grid=(B,),
        in_specs=in_specs,
        out_specs=out_specs,
        out_shape=out_shape,
    )(*args)


def _mlp_body(*refs, K, has_w1):
    if has_w1:
        (g_ref, bias_ref, w1_ref, w2_ref, b2_ref, w3_ref, b3_ref,
         out_ref) = refs
        x = jnp.dot(g_ref[...], w1_ref[...],
                    preferred_element_type=F32)
    else:
        g_ref, bias_ref, w2_ref, b2_ref, w3_ref, b3_ref, out_ref = refs
        x = g_ref[...]
    BSK, O1 = x.shape
    BS = BSK // K
    x = x.reshape(BS, K, O1) + bias_ref[...][:, None, :]
    x = jnp.maximum(x, 0.0).reshape(BSK, O1)
    x = jnp.maximum(jnp.dot(x, w2_ref[...], preferred_element_type=F32)
                    + b2_ref[...], 0.0)
    x = jnp.maximum(jnp.dot(x, w3_ref[...], preferred_element_type=F32)
                    + b3_ref[...], 0.0)
    out_ref[...] = jnp.max(x.reshape(BS, K, -1), axis=1)


def _mlp(rows, bias, w1, w2, b2, w3, b3, K, BS):
    R, D = rows.shape
    M, O1 = bias.shape
    O2, O3 = w2.shape[1], w3.shape[1]
    i0 = lambda i: (i, 0)
    k2 = lambda i: (0, 0)
    in_specs = [pl.BlockSpec((BS * K, D), i0), pl.BlockSpec((BS, O1), i0)]
    args = [rows, bias]
    if w1 is not None:
        in_specs.append(pl.BlockSpec((D, O1), k2))
        args.append(w1)
    in_specs += [pl.BlockSpec((O1, O2), k2), pl.BlockSpec((1, O2), k2),
                 pl.BlockSpec((O2, O3), k2), pl.BlockSpec((1, O3), k2)]
    args += [w2, b2, w3, b3]
    return pl.pallas_call(
        functools.partial(_mlp_body, K=K, has_w1=w1 is not None),
        grid=(M // BS,),
        in_specs=in_specs,
        out_specs=pl.BlockSpec((BS, O3), i0),
        out_shape=jax.ShapeDtypeStruct((M, O3), F32),
    )(*args)


def _sa3_body(x_ref, w1, b1, w2, b2, w3, b3, out_ref, *, B, P):
    x = x_ref[...].reshape(B * P, -1)
    x = jnp.maximum(jnp.dot(x, w1[...], preferred_element_type=F32) + b1[...], 0.0)
    x = jnp.maximum(jnp.dot(x, w2[...], preferred_element_type=F32) + b2[...], 0.0)
    x = jnp.maximum(jnp.dot(x, w3[...], preferred_element_type=F32) + b3[...], 0.0)
    out_ref[...] = jnp.max(x.reshape(B, P, -1), axis=1)


def _sa3(feats, w1, b1, w2, b2, w3, b3):
    B, P, Cp = feats.shape
    O3 = w3.shape[1]
    return pl.pallas_call(
        functools.partial(_sa3_body, B=B, P=P),
        out_shape=jax.ShapeDtypeStruct((B, O3), F32),
    )(feats, w1, b1, w2, b2, w3, b3)


def _pad_rows(w, rows):
    return jnp.pad(w, ((0, rows - w.shape[0]), (0, 0)))


def _stage_weights(scale_params, c_pts, cp):
    out = []
    for layers in scale_params:
        w1t, b1 = _fold(layers[0])
        w2t, b2 = _fold(layers[1])
        w3t, b3 = _fold(layers[2])
        w1p = _pad_rows(w1t, cp)
        wb = jnp.concatenate([-w1t[c_pts:c_pts + 3], b1[None, :]], axis=0)
        wb = _pad_rows(wb, 8)
        out.append((w1p, wb, w2t, b2[None, :], w3t, b3[None, :]))
    return out


_SC_CORES, _SC_SUBCORES = 2, 16
_SC_WORKERS = _SC_CORES * _SC_SUBCORES
_CH = 128


def _sc_gather_body(rpw, gr, ngr, nbuf, table_hbm, idx_hbm, out_hbm,
                    idx_v, *rest):
    bufs = rest[:nbuf]
    ssems = rest[nbuf:2 * nbuf]
    gsem = rest[2 * nbuf]
    wid = lax.axis_index("s") * _SC_CORES + lax.axis_index("c")
    base = wid * rpw
    ncpg = gr // _CH

    pltpu.sync_copy(idx_hbm.at[pl.ds(base, rpw)], idx_v)

    def do_group(g, buf, ssem):
        cps = []
        for c in range(ncpg):
            cps.append(pltpu.async_copy(
                table_hbm.at[idx_v.at[pl.ds(g * gr + c * _CH, _CH)]],
                buf.at[pl.ds(c * _CH, _CH)], gsem))
        for cp in cps:
            cp.wait()
        pltpu.async_copy(buf, out_hbm.at[pl.ds(base + g * gr, gr)], ssem)

    def drain_store(buf, ssem):
        pltpu.make_async_copy(out_hbm.at[pl.ds(0, gr)], buf, ssem).wait()

    nprime = min(nbuf, ngr)
    for g in range(nprime):
        do_group(g, bufs[g], ssems[g])
    if ngr > nbuf:

        def body(g, carry):
            b = lax.rem(g, nbuf)
            for k in range(nbuf):

                @pl.when(b == k)
                def _(k=k):
                    drain_store(bufs[k], ssems[k])
                    do_group(g, bufs[k], ssems[k])

            return carry

        lax.fori_loop(nbuf, ngr, body, 0)
    for g in range(nprime):
        drain_store(bufs[g], ssems[g])


def _sc_gather_rows(table, idx):
    _, D = table.shape
    R = idx.shape[0]
    rpw = R // _SC_WORKERS
    gr = min(rpw, max(_CH, 32768 // D))
    ngr = rpw // gr
    nbuf = min(3, ngr)
    mesh = plsc.VectorSubcoreMesh(core_axis_name="c", subcore_axis_name="s")
    return pl.kernel(
        functools.partial(_sc_gather_body, rpw, gr, ngr, nbuf),
        mesh=mesh,
        out_type=jax.ShapeDtypeStruct((R, D), F32),
        scratch_types=(
            [pltpu.VMEM((rpw,), jnp.int32)]
            + [pltpu.VMEM((gr, D), F32) for _ in range(nbuf)]
            + [pltpu.SemaphoreType.DMA for _ in range(nbuf + 1)]
        ),
        compiler_params=pltpu.CompilerParams(use_tc_tiling_on_sc=False),
    )(table, idx)


def _sc_select_body(rpw, N, K, S, cr, tgt_hbm, init_hbm,
                    out_hbm, out_v, tgt_v, gsem):
    wid = lax.axis_index("s") * _SC_CORES + lax.axis_index("c")
    row0 = wid * rpw
    base_slot = row0 * K
    slab = rpw * K
    voff = (row0 // S) * N
    gpr = N // 16
    nch = rpw // cr

    pltpu.sync_copy(init_hbm.at[pl.ds(base_slot, slab)], out_v)
    lanes = lax.iota(jnp.int32, 16)

    def chunk(c, carry):
        pltpu.sync_copy(tgt_hbm.at[pl.ds((row0 + c * cr) * N, cr * N)], tgt_v)

        def group(i, carry2):
            t = lax.rem(i, gpr)
            tg = tgt_v[pl.ds(i * 16, 16)] - base_slot
            msk = tg < slab
            val = lanes + (t * 16 + voff)
            plsc.store_scatter(out_v, (tg,), val, mask=msk)
            return carry2

        lax.fori_loop(0, cr * gpr, group, 0)
        return carry

    lax.fori_loop(0, nch, chunk, 0)
    pltpu.sync_copy(out_v, out_hbm.at[pl.ds(base_slot, slab)])


def _sc_select(tgt, init, N, K, S):
    M = tgt.shape[0]
    rpw = M // _SC_WORKERS
    cr = max(1, (16 * 1024) // N)
    cr = min(cr, rpw)
    return pl.kernel(
        functools.partial(_sc_select_body, rpw, N, K, S, cr),
        mesh=plsc.VectorSubcoreMesh(core_axis_name="c", subcore_axis_name="s"),
        out_type=jax.ShapeDtypeStruct((M * K,), jnp.int32),
        scratch_types=[
            pltpu.VMEM((rpw * K,), jnp.int32),
            pltpu.VMEM((cr * N,), jnp.int32),
            pltpu.SemaphoreType.DMA,
        ],
        compiler_params=pltpu.CompilerParams(needs_layout_passes=False),
    )(tgt.reshape(-1), init.reshape(-1))


def _sa_msg(feats, naug, nxyz_b3s, weights, r2s, Ks, S, BS, raw_gather):
    B, N, Cp = feats.shape
    lt = jnp.triu(jnp.ones((N, N), F32))
    w1s = [w[0] for w in weights]
    wbs = [w[1] for w in weights]
    o1s = [w[0].shape[1] for w in weights]
    nscale = len(Ks)
    res = _ball(naug, nxyz_b3s, feats, lt, w1s, wbs, o1s, r2s, Ks,
                emit_g=not raw_gather)
    inits = res[:nscale]
    tgts = res[nscale:2 * nscale]
    if raw_gather:
        biases = res[2 * nscale:]
        gs = [None] * nscale
        table = feats.reshape(B * N, Cp)
    else:
        gs = res[2 * nscale:3 * nscale]
        biases = res[3 * nscale:]
        table = None
    idxs = [_sc_select(tgts[i].reshape(B * S, N), inits[i], N, Ks[i], S)
            for i in range(nscale)]
    rows = [_sc_gather_rows(table if raw_gather else gs[i].reshape(B * N, -1),
                            idxs[i]) for i in range(nscale)]
    outs = []
    for i in range(nscale):
        w1, _, w2, b2, w3, b3 = weights[i]
        o = _mlp(rows[i], biases[i].reshape(B * S, -1),
                 w1 if raw_gather else None, w2, b2, w3, b3, Ks[i], BS)
        outs.append(o.reshape(B, S, -1))
    return outs


def kernel(xyz, params):
    B, C6, N = xyz.shape
    coords = xyz[:, :3, :]
    norm = xyz[:, 3:, :]

    S1, Ks1 = 512, (16, 32, 128)
    r2s1 = tuple(np.float32(float(r) ** 2) for r in (0.1, 0.2, 0.4))
    fps_in = jnp.transpose(coords, (1, 0, 2))
    c1 = _fps(fps_in, S1)
    nxyz1 = jnp.transpose(c1, (1, 2, 0))
    naug1 = jnp.concatenate(
        [nxyz1, jnp.ones((B, S1, 1), F32), jnp.zeros((B, S1, 4), F32)], -1)
    feats1 = jnp.concatenate(
        [jnp.transpose(norm, (0, 2, 1)), jnp.transpose(coords, (0, 2, 1)),
         jnp.zeros((B, N, 10), F32)], -1)
    w_s1 = _stage_weights(params['sa1'], 3, 16)
    outs1 = _sa_msg(feats1, naug1, coords, w_s1, r2s1, Ks1, S1, BS=64,
                    raw_gather=True)
    l1_points = jnp.concatenate(outs1, -1)

    S2, Ks2 = 128, (32, 64, 128)
    r2s2 = tuple(np.float32(float(r) ** 2) for r in (0.2, 0.4, 0.8))
    c2 = _fps(c1, S2)
    nxyz2 = jnp.transpose(c2, (1, 2, 0))
    naug2 = jnp.concatenate(
        [nxyz2, jnp.ones((B, S2, 1), F32), jnp.zeros((B, S2, 4), F32)], -1)
    l1_xyz_b3s = jnp.transpose(c1, (1, 0, 2))
    feats2 = jnp.concatenate(
        [l1_points, nxyz1, jnp.zeros((B, S1, 384 - 323), F32)], -1)
    w_s2 = _stage_weights(params['sa2'], 320, 384)
    outs2 = _sa_msg(feats2, naug2, l1_xyz_b3s, w_s2, r2s2, Ks2, S2, BS=32,
                    raw_gather=False)
    l2_points = jnp.concatenate(outs2, -1)

    w1t, b1 = _fold(params['sa3'][0])
    w2t, b2 = _fold(params['sa3'][1])
    w3t, b3 = _fold(params['sa3'][2])
    cp3 = 768
    feats3 = jnp.concatenate(
        [nxyz2, l2_points, jnp.zeros((B, S2, cp3 - 643), F32)], -1)
    out = _sa3(feats3, _pad_rows(w1t, cp3), b1[None, :], w2t, b2[None, :],
               w3t, b3[None, :])
    return out


# speedup vs baseline: 1.0464x; 1.0464x over previous
import functools

import jax
import jax.numpy as jnp
import numpy as np
from jax import lax
from jax.experimental import pallas as pl
from jax.experimental.pallas import tpu as pltpu
from jax.experimental.pallas import tpu_sc as plsc

F32 = jnp.float32


def _fold(layer):
    s = layer['gamma'] / jnp.sqrt(layer['var'] + 1e-5)
    w = layer['W'] * s[:, None]
    b = (layer['b'] - layer['mean']) * s + layer['beta']
    return w.T, b


def _fps_body(xyz_ref, out_ref, *, npoint):
    xyz = xyz_ref[...]
    _, B, N = xyz.shape
    lane_n = jax.lax.broadcasted_iota(jnp.int32, (B, N), 1)
    lane_s = jax.lax.broadcasted_iota(jnp.int32, (3, B, npoint), 2)

    def body(i, st):
        dist, far, acc = st
        onehot = (lane_n == far).astype(F32)
        c = jnp.sum(xyz * onehot[None, :, :], axis=2, keepdims=True)
        acc = jnp.where(lane_s == i, c, acc)
        diff = xyz - c
        d = jnp.sum(diff * diff, axis=0)
        dist = jnp.minimum(dist, d)
        m = jnp.max(dist, axis=1, keepdims=True)
        far = jnp.min(jnp.where(dist == m, lane_n, N), axis=1, keepdims=True)
        return dist, far, acc

    dist0 = jnp.full((B, N), 1e10, F32)
    far0 = jnp.zeros((B, 1), jnp.int32)
    acc0 = jnp.zeros((3, B, npoint), F32)
    _, _, acc = jax.lax.fori_loop(0, npoint, body, (dist0, far0, acc0))
    out_ref[...] = acc


def _fps(xyz3bn, npoint):
    _, B, N = xyz3bn.shape
    return pl.pallas_call(
        functools.partial(_fps_body, npoint=npoint),
        out_shape=jax.ShapeDtypeStruct((3, B, npoint), F32),
    )(xyz3bn)


def _ball_body(naug_ref, xyz3_ref, feats_ref, lt_ref, *rest, r2s, Ks, N, S,
               emit_g):
    nscale = len(Ks)
    if emit_g:
        w1s = rest[:nscale]
        wbs = rest[nscale:2 * nscale]
        idx_refs = rest[2 * nscale:3 * nscale]
        tgt_refs = rest[3 * nscale:4 * nscale]
        g_refs = rest[4 * nscale:5 * nscale]
        bias_refs = rest[5 * nscale:6 * nscale]
    else:
        w1s = g_refs = None
        wbs = rest[:nscale]
        idx_refs = rest[nscale:2 * nscale]
        tgt_refs = rest[2 * nscale:3 * nscale]
        bias_refs = rest[3 * nscale:4 * nscale]

    b = pl.program_id(0)
    naug = naug_ref[0]
    x3 = xyz3_ref[0]
    lt = lt_ref[...]

    nxyz = naug[:, :3]
    sc = jnp.sum(nxyz * nxyz, axis=1, keepdims=True)
    sx = jnp.sum(x3 * x3, axis=0, keepdims=True)
    cross = jnp.dot(nxyz, x3, preferred_element_type=F32)
    sq = (sc + sx) - 2.0 * cross

    lane_j = jax.lax.broadcasted_iota(jnp.int32, (S, N), 1)
    row_s = jax.lax.broadcasted_iota(jnp.int32, (S, N), 0)
    M = pl.num_programs(0) * S
    for i in range(nscale):
        K = Ks[i]
        validb = sq <= r2s[i]
        valid = validb.astype(F32)
        rc = jnp.dot(valid, lt, preferred_element_type=F32)
        inr = jnp.logical_and(validb, rc <= float(K))
        tgt = jnp.where(inr,
                        (b * S + row_s) * K + (rc.astype(jnp.int32) - 1),
                        M * K)
        tgt_refs[i][0] = tgt
        first = jnp.min(jnp.where(validb, lane_j, N), axis=1, keepdims=True)
        idx_refs[i][0] = jnp.broadcast_to(first + b * N, (S, K))
        if emit_g:
            g_refs[i][0] = jnp.dot(feats_ref[0], w1s[i][...],
                                   preferred_element_type=F32)
        bias_refs[i][0] = jnp.dot(naug, wbs[i][...],
                                  preferred_element_type=F32)


def _ball(naug, xyz3, feats, lt, w1s, wbs, o1s, r2s, Ks, emit_g):
    B, S, _ = naug.shape
    N = xyz3.shape[2]
    Cp = feats.shape[2]
    nscale = len(Ks)
    c0 = lambda i: (i, 0, 0)
    k2 = lambda i: (0, 0)
    in_specs = [
        pl.BlockSpec((1, S, 8), c0),
        pl.BlockSpec((1, 3, N), c0),
        pl.BlockSpec((1, N, Cp), c0),
        pl.BlockSpec((N, N), k2),
    ]
    if emit_g:
        in_specs += [pl.BlockSpec((Cp, o), k2) for o in o1s]
    in_specs += [pl.BlockSpec((8, o), k2) for o in o1s]
    out_specs = [pl.BlockSpec((1, S, Ks[i]), c0) for i in range(nscale)]
    out_shape = [jax.ShapeDtypeStruct((B, S, Ks[i]), jnp.int32)
                 for i in range(nscale)]
    out_specs += [pl.BlockSpec((1, S, N), c0) for _ in range(nscale)]
    out_shape += [jax.ShapeDtypeStruct((B, S, N), jnp.int32)
                  for _ in range(nscale)]
    if emit_g:
        out_specs += [pl.BlockSpec((1, N, o), c0) for o in o1s]
        out_shape += [jax.ShapeDtypeStruct((B, N, o), F32) for o in o1s]
    out_specs += [pl.BlockSpec((1, S, o), c0) for o in o1s]
    out_shape += [jax.ShapeDtypeStruct((B, S, o), F32) for o in o1s]
    args = [naug, xyz3, feats, lt] + (list(w1s) if emit_g else []) + list(wbs)
    return pl.pallas_call(
        functools.partial(_ball_body, r2s=tuple(r2s), Ks=tuple(Ks), N=N, S=S,
                          emit_g=emit_g),
        grid=(B,),
        in_specs=in_specs,
        out_specs=out_specs,
        out_shape=out_shape,
    )(*args)


def _mlp_body(*refs, K, has_w1):
    if has_w1:
        (g_ref, bias_ref, w1_ref, w2_ref, b2_ref, w3_ref, b3_ref,
         out_ref) = refs
        x = jnp.dot(g_ref[...], w1_ref[...],
                    preferred_element_type=F32)
    else:
        g_ref, bias_ref, w2_ref, b2_ref, w3_ref, b3_ref, out_ref = refs
        x = g_ref[...]
    BSK, O1 = x.shape
    BS = BSK // K
    x = x.reshape(BS, K, O1) + bias_ref[...][:, None, :]
    x = jnp.maximum(x, 0.0).reshape(BSK, O1)
    x = jnp.maximum(jnp.dot(x, w2_ref[...], preferred_element_type=F32)
                    + b2_ref[...], 0.0)
    x = jnp.maximum(jnp.dot(x, w3_ref[...], preferred_element_type=F32)
                    + b3_ref[...], 0.0)
    out_ref[...] = jnp.max(x.reshape(BS, K, -1), axis=1)


def _mlp(rows, bias, w1, w2, b2, w3, b3, K, BS):
    R, D = rows.shape
    M, O1 = bias.shape
    O2, O3 = w2.shape[1], w3.shape[1]
    i0 = lambda i: (i, 0)
    k2 = lambda i: (0, 0)
    in_specs = [pl.BlockSpec((BS * K, D), i0), pl.BlockSpec((BS, O1), i0)]
    args = [rows, bias]
    if w1 is not None:
        in_specs.append(pl.BlockSpec((D, O1), k2))
        args.append(w1)
    in_specs += [pl.BlockSpec((O1, O2), k2), pl.BlockSpec((1, O2), k2),
                 pl.BlockSpec((O2, O3), k2), pl.BlockSpec((1, O3), k2)]
    args += [w2, b2, w3, b3]
    return pl.pallas_call(
        functools.partial(_mlp_body, K=K, has_w1=w1 is not None),
        grid=(M // BS,),
        in_specs=in_specs,
        out_specs=pl.BlockSpec((BS, O3), i0),
        out_shape=jax.ShapeDtypeStruct((M, O3), F32),
    )(*args)


def _sa3_body(x_ref, w1, b1, w2, b2, w3, b3, out_ref, *, B, P):
    x = x_ref[...].reshape(B * P, -1)
    x = jnp.maximum(jnp.dot(x, w1[...], preferred_element_type=F32) + b1[...], 0.0)
    x = jnp.maximum(jnp.dot(x, w2[...], preferred_element_type=F32) + b2[...], 0.0)
    x = jnp.maximum(jnp.dot(x, w3[...], preferred_element_type=F32) + b3[...], 0.0)
    out_ref[...] = jnp.max(x.reshape(B, P, -1), axis=1)


def _sa3(feats, w1, b1, w2, b2, w3, b3):
    B, P, Cp = feats.shape
    O3 = w3.shape[1]
    return pl.pallas_call(
        functools.partial(_sa3_body, B=B, P=P),
        out_shape=jax.ShapeDtypeStruct((B, O3), F32),
    )(feats, w1, b1, w2, b2, w3, b3)


def _pad_rows(w, rows):
    return jnp.pad(w, ((0, rows - w.shape[0]), (0, 0)))


def _stage_weights(scale_params, c_pts, cp):
    out = []
    for layers in scale_params:
        w1t, b1 = _fold(layers[0])
        w2t, b2 = _fold(layers[1])
        w3t, b3 = _fold(layers[2])
        w1p = _pad_rows(w1t, cp)
        wb = jnp.concatenate([-w1t[c_pts:c_pts + 3], b1[None, :]], axis=0)
        wb = _pad_rows(wb, 8)
        out.append((w1p, wb, w2t, b2[None, :], w3t, b3[None, :]))
    return out


_SC_CORES, _SC_SUBCORES = 2, 16
_SC_WORKERS = _SC_CORES * _SC_SUBCORES
_CH = 128


def _sc_gather_body(rpw, gr, ngr, nbuf, table_hbm, idx_hbm, out_hbm,
                    idx_v, *rest):
    bufs = rest[:nbuf]
    ssems = rest[nbuf:2 * nbuf]
    gsem = rest[2 * nbuf]
    wid = lax.axis_index("s") * _SC_CORES + lax.axis_index("c")
    base = wid * rpw
    ncpg = gr // _CH

    pltpu.sync_copy(idx_hbm.at[pl.ds(base, rpw)], idx_v)

    def do_group(g, buf, ssem):
        cps = []
        for c in range(ncpg):
            cps.append(pltpu.async_copy(
                table_hbm.at[idx_v.at[pl.ds(g * gr + c * _CH, _CH)]],
                buf.at[pl.ds(c * _CH, _CH)], gsem))
        for cp in cps:
            cp.wait()
        pltpu.async_copy(buf, out_hbm.at[pl.ds(base + g * gr, gr)], ssem)

    def drain_store(buf, ssem):
        pltpu.make_async_copy(out_hbm.at[pl.ds(0, gr)], buf, ssem).wait()

    nprime = min(nbuf, ngr)
    for g in range(nprime):
        do_group(g, bufs[g], ssems[g])
    if ngr > nbuf:

        def body(g, carry):
            b = lax.rem(g, nbuf)
            for k in range(nbuf):

                @pl.when(b == k)
                def _(k=k):
                    drain_store(bufs[k], ssems[k])
                    do_group(g, bufs[k], ssems[k])

            return carry

        lax.fori_loop(nbuf, ngr, body, 0)
    for g in range(nprime):
        drain_store(bufs[g], ssems[g])


def _sc_gather_rows(table, idx):
    _, D = table.shape
    R = idx.shape[0]
    rpw = R // _SC_WORKERS
    gr = min(rpw, max(_CH, 32768 // D))
    ngr = rpw // gr
    nbuf = min(3, ngr)
    mesh = plsc.VectorSubcoreMesh(core_axis_name="c", subcore_axis_name="s")
    return pl.kernel(
        functools.partial(_sc_gather_body, rpw, gr, ngr, nbuf),
        mesh=mesh,
        out_type=jax.ShapeDtypeStruct((R, D), F32),
        scratch_types=(
            [pltpu.VMEM((rpw,), jnp.int32)]
            + [pltpu.VMEM((gr, D), F32) for _ in range(nbuf)]
            + [pltpu.SemaphoreType.DMA for _ in range(nbuf + 1)]
        ),
        compiler_params=pltpu.CompilerParams(use_tc_tiling_on_sc=False),
    )(table, idx)


def _sc_select_body(rpw, N, K, S, cr, tgt_hbm, init_hbm,
                    out_hbm, out_v, tgt_v, gsem):
    wid = lax.axis_index("s") * _SC_CORES + lax.axis_index("c")
    row0 = wid * rpw
    base_slot = row0 * K
    slab = rpw * K
    voff = (row0 // S) * N
    gpr = N // 16
    nch = rpw // cr

    pltpu.sync_copy(init_hbm.at[pl.ds(base_slot, slab)], out_v)
    lanes = lax.iota(jnp.int32, 16)

    def chunk(c, carry):
        pltpu.sync_copy(tgt_hbm.at[pl.ds((row0 + c * cr) * N, cr * N)], tgt_v)

        def group(i, carry2):
            t = lax.rem(i, gpr)
            tg = tgt_v[pl.ds(i * 16, 16)] - base_slot
            msk = tg < slab
            val = lanes + (t * 16 + voff)
            plsc.store_scatter(out_v, (tg,), val, mask=msk)
            return carry2

        lax.fori_loop(0, cr * gpr, group, 0)
        return carry

    lax.fori_loop(0, nch, chunk, 0)
    pltpu.sync_copy(out_v, out_hbm.at[pl.ds(base_slot, slab)])


def _sc_select(tgt, init, N, K, S):
    M = tgt.shape[0]
    rpw = M // _SC_WORKERS
    cr = max(1, (16 * 1024) // N)
    cr = min(cr, rpw)
    return pl.kernel(
        functools.partial(_sc_select_body, rpw, N, K, S, cr),
        mesh=plsc.VectorSubcoreMesh(core_axis_name="c", subcore_axis_name="s"),
        out_type=jax.ShapeDtypeStruct((M * K,), jnp.int32),
        scratch_types=[
            pltpu.VMEM((rpw * K,), jnp.int32),
            pltpu.VMEM((cr * N,), jnp.int32),
            pltpu.SemaphoreType.DMA,
        ],
        compiler_params=pltpu.CompilerParams(needs_layout_passes=False),
    )(tgt.reshape(-1), init.reshape(-1))


def _sc_select_gather_body(rpw, N, K, S, cr, gr, ngr, nbuf, tgt_hbm, init_hbm,
                           table_hbm, out_hbm, idx_v, tgt_v, *rest):
    bufs = rest[:nbuf]
    ssems = rest[nbuf:2 * nbuf]
    gsem = rest[2 * nbuf]
    wid = lax.axis_index("s") * _SC_CORES + lax.axis_index("c")
    row0 = wid * rpw
    base_slot = row0 * K
    slab = rpw * K
    voff = (row0 // S) * N
    gpr = N // 16
    nch = rpw // cr
    ncpg = gr // _CH

    pltpu.sync_copy(init_hbm.at[pl.ds(base_slot, slab)], idx_v)
    lanes = lax.iota(jnp.int32, 16)

    def chunk(c, carry):
        pltpu.sync_copy(tgt_hbm.at[pl.ds((row0 + c * cr) * N, cr * N)], tgt_v)

        def group(i, carry2):
            t = lax.rem(i, gpr)
            tg = tgt_v[pl.ds(i * 16, 16)] - base_slot
            msk = tg < slab
            val = lanes + (t * 16 + voff)
            plsc.store_scatter(idx_v, (tg,), val, mask=msk)
            return carry2

        lax.fori_loop(0, cr * gpr, group, 0)
        return carry

    lax.fori_loop(0, nch, chunk, 0)

    def do_group(g, buf, ssem):
        cps = []
        for c in range(ncpg):
            cps.append(pltpu.async_copy(
                table_hbm.at[idx_v.at[pl.ds(g * gr + c * _CH, _CH)]],
                buf.at[pl.ds(c * _CH, _CH)], gsem))
        for cp in cps:
            cp.wait()
        pltpu.async_copy(buf, out_hbm.at[pl.ds(base_slot + g * gr, gr)], ssem)

    def drain_store(buf, ssem):
        pltpu.make_async_copy(out_hbm.at[pl.ds(0, gr)], buf, ssem).wait()

    nprime = min(nbuf, ngr)
    for g in range(nprime):
        do_group(g, bufs[g], ssems[g])
    if ngr > nbuf:

        def body(g, carry):
            b = lax.rem(g, nbuf)
            for k in range(nbuf):

                @pl.when(b == k)
                def _(k=k):
                    drain_store(bufs[k], ssems[k])
                    do_group(g, bufs[k], ssems[k])

            return carry

        lax.fori_loop(nbuf, ngr, body, 0)
    for g in range(nprime):
        drain_store(bufs[g], ssems[g])


def _sc_select_gather(tgt, init, table, N, K, S):
    M = tgt.shape[0]
    _, D = table.shape
    rpw = M // _SC_WORKERS
    slab = rpw * K
    gr = min(slab, max(_CH, 32768 // D))
    ngr = slab // gr
    nbuf = min(2, ngr)
    cr = min(max(1, (16 * 1024) // N), rpw)
    return pl.kernel(
        functools.partial(_sc_select_gather_body, rpw, N, K, S, cr, gr, ngr,
                          nbuf),
        mesh=plsc.VectorSubcoreMesh(core_axis_name="c", subcore_axis_name="s"),
        out_type=jax.ShapeDtypeStruct((M * K, D), F32),
        scratch_types=(
            [pltpu.VMEM((slab,), jnp.int32), pltpu.VMEM((cr * N,), jnp.int32)]
            + [pltpu.VMEM((gr, D), F32) for _ in range(nbuf)]
            + [pltpu.SemaphoreType.DMA for _ in range(nbuf + 1)]
        ),
        compiler_params=pltpu.CompilerParams(use_tc_tiling_on_sc=False,
                                             needs_layout_passes=False),
    )(tgt.reshape(-1), init.reshape(-1), table)


def _sa_msg(feats, naug, nxyz_b3s, weights, r2s, Ks, S, BS, raw_gather):
    B, N, Cp = feats.shape
    lt = jnp.triu(jnp.ones((N, N), F32))
    w1s = [w[0] for w in weights]
    wbs = [w[1] for w in weights]
    o1s = [w[0].shape[1] for w in weights]
    nscale = len(Ks)
    res = _ball(naug, nxyz_b3s, feats, lt, w1s, wbs, o1s, r2s, Ks,
                emit_g=not raw_gather)
    inits = res[:nscale]
    tgts = res[nscale:2 * nscale]
    if raw_gather:
        biases = res[2 * nscale:]
        gs = [None] * nscale
        table = feats.reshape(B * N, Cp)
    else:
        gs = res[2 * nscale:3 * nscale]
        biases = res[3 * nscale:]
        table = None
    rows = [_sc_select_gather(
        tgts[i].reshape(B * S, N), inits[i],
        table if raw_gather else gs[i].reshape(B * N, -1), N, Ks[i], S)
        for i in range(nscale)]
    outs = []
    for i in range(nscale):
        w1, _, w2, b2, w3, b3 = weights[i]
        o = _mlp(rows[i], biases[i].reshape(B * S, -1),
                 w1 if raw_gather else None, w2, b2, w3, b3, Ks[i], BS)
        outs.append(o.reshape(B, S, -1))
    return outs


def kernel(xyz, params):
    B, C6, N = xyz.shape
    coords = xyz[:, :3, :]
    norm = xyz[:, 3:, :]

    S1, Ks1 = 512, (16, 32, 128)
    r2s1 = tuple(np.float32(float(r) ** 2) for r in (0.1, 0.2, 0.4))
    fps_in = jnp.transpose(coords, (1, 0, 2))
    c1 = _fps(fps_in, S1)
    nxyz1 = jnp.transpose(c1, (1, 2, 0))
    naug1 = jnp.concatenate(
        [nxyz1, jnp.ones((B, S1, 1), F32), jnp.zeros((B, S1, 4), F32)], -1)
    feats1 = jnp.concatenate(
        [jnp.transpose(norm, (0, 2, 1)), jnp.transpose(coords, (0, 2, 1)),
         jnp.zeros((B, N, 10), F32)], -1)
    w_s1 = _stage_weights(params['sa1'], 3, 16)
    outs1 = _sa_msg(feats1, naug1, coords, w_s1, r2s1, Ks1, S1, BS=64,
                    raw_gather=True)
    l1_points = jnp.concatenate(outs1, -1)

    S2, Ks2 = 128, (32, 64, 128)
    r2s2 = tuple(np.float32(float(r) ** 2) for r in (0.2, 0.4, 0.8))
    c2 = _fps(c1, S2)
    nxyz2 = jnp.transpose(c2, (1, 2, 0))
    naug2 = jnp.concatenate(
        [nxyz2, jnp.ones((B, S2, 1), F32), jnp.zeros((B, S2, 4), F32)], -1)
    l1_xyz_b3s = jnp.transpose(c1, (1, 0, 2))
    feats2 = jnp.concatenate(
        [l1_points, nxyz1, jnp.zeros((B, S1, 384 - 323), F32)], -1)
    w_s2 = _stage_weights(params['sa2'], 320, 384)
    outs2 = _sa_msg(feats2, naug2, l1_xyz_b3s, w_s2, r2s2, Ks2, S2, BS=32,
                    raw_gather=False)
    l2_points = jnp.concatenate(outs2, -1)

    w1t, b1 = _fold(params['sa3'][0])
    w2t, b2 = _fold(params['sa3'][1])
    w3t, b3 = _fold(params['sa3'][2])
    cp3 = 768
    feats3 = jnp.concatenate(
        [nxyz2, l2_points, jnp.zeros((B, S2, cp3 - 643), F32)], -1)
    out = _sa3(feats3, _pad_rows(w1t, cp3), b1[None, :], w2t, b2[None, :],
               w3t, b3[None, :])
    return out


# speedup vs baseline: 1.0464x; 1.0000x over previous
import functools

import jax
import jax.numpy as jnp
import numpy as np
from jax import lax
from jax.experimental import pallas as pl
from jax.experimental.pallas import tpu as pltpu
from jax.experimental.pallas import tpu_sc as plsc

F32 = jnp.float32


def _fold(layer):
    s = layer['gamma'] / jnp.sqrt(layer['var'] + 1e-5)
    w = layer['W'] * s[:, None]
    b = (layer['b'] - layer['mean']) * s + layer['beta']
    return w.T, b


def _fps_body(xyz_ref, out_ref, *, npoint):
    xyz = xyz_ref[...]
    _, B, N = xyz.shape
    lane_n = jax.lax.broadcasted_iota(jnp.int32, (B, N), 1)
    lane_s = jax.lax.broadcasted_iota(jnp.int32, (3, B, npoint), 2)

    def body(i, st):
        dist, far, acc = st
        onehot = (lane_n == far).astype(F32)
        c = jnp.sum(xyz * onehot[None, :, :], axis=2, keepdims=True)
        acc = jnp.where(lane_s == i, c, acc)
        diff = xyz - c
        d = jnp.sum(diff * diff, axis=0)
        dist = jnp.minimum(dist, d)
        m = jnp.max(dist, axis=1, keepdims=True)
        far = jnp.min(jnp.where(dist == m, lane_n, N), axis=1, keepdims=True)
        return dist, far, acc

    dist0 = jnp.full((B, N), 1e10, F32)
    far0 = jnp.zeros((B, 1), jnp.int32)
    acc0 = jnp.zeros((3, B, npoint), F32)
    _, _, acc = jax.lax.fori_loop(0, npoint, body, (dist0, far0, acc0))
    out_ref[...] = acc


def _fps(xyz3bn, npoint):
    _, B, N = xyz3bn.shape
    return pl.pallas_call(
        functools.partial(_fps_body, npoint=npoint),
        out_shape=jax.ShapeDtypeStruct((3, B, npoint), F32),
    )(xyz3bn)


def _ball_body(naug_ref, xyz3_ref, feats_ref, lt_ref, *rest, r2s, Ks, N, S,
               emit_g):
    nscale = len(Ks)
    if emit_g:
        w1s = rest[:nscale]
        wbs = rest[nscale:2 * nscale]
        idx_refs = rest[2 * nscale:3 * nscale]
        tgt_refs = rest[3 * nscale:4 * nscale]
        g_refs = rest[4 * nscale:5 * nscale]
        bias_refs = rest[5 * nscale:6 * nscale]
    else:
        w1s = g_refs = None
        wbs = rest[:nscale]
        idx_refs = rest[nscale:2 * nscale]
        tgt_refs = rest[2 * nscale:3 * nscale]
        bias_refs = rest[3 * nscale:4 * nscale]

    b = pl.program_id(0)
    naug = naug_ref[0]
    x3 = xyz3_ref[0]
    lt = lt_ref[...]

    nxyz = naug[:, :3]
    sc = jnp.sum(nxyz * nxyz, axis=1, keepdims=True)
    sx = jnp.sum(x3 * x3, axis=0, keepdims=True)
    cross = jnp.dot(nxyz, x3, preferred_element_type=F32)
    sq = (sc + sx) - 2.0 * cross

    lane_j = jax.lax.broadcasted_iota(jnp.int32, (S, N), 1)
    row_s = jax.lax.broadcasted_iota(jnp.int32, (S, N), 0)
    M = pl.num_programs(0) * S
    for i in range(nscale):
        K = Ks[i]
        validb = sq <= r2s[i]
        valid = validb.astype(F32)
        rc = jnp.dot(valid, lt, preferred_element_type=F32)
        inr = jnp.logical_and(validb, rc <= float(K))
        tgt = jnp.where(inr,
                        (b * S + row_s) * K + (rc.astype(jnp.int32) - 1),
                        M * K)
        tgt_refs[i][0] = tgt
        first = jnp.min(jnp.where(validb, lane_j, N), axis=1, keepdims=True)
        idx_refs[i][0] = jnp.broadcast_to(first + b * N, (S, K))
        if emit_g:
            g_refs[i][0] = jnp.dot(feats_ref[0], w1s[i][...],
                                   preferred_element_type=F32)
        bias_refs[i][0] = jnp.dot(naug, wbs[i][...],
                                  preferred_element_type=F32)


def _ball(naug, xyz3, feats, lt, w1s, wbs, o1s, r2s, Ks, emit_g):
    B, S, _ = naug.shape
    N = xyz3.shape[2]
    Cp = feats.shape[2]
    nscale = len(Ks)
    c0 = lambda i: (i, 0, 0)
    k2 = lambda i: (0, 0)
    in_specs = [
        pl.BlockSpec((1, S, 8), c0),
        pl.BlockSpec((1, 3, N), c0),
        pl.BlockSpec((1, N, Cp), c0),
        pl.BlockSpec((N, N), k2),
    ]
    if emit_g:
        in_specs += [pl.BlockSpec((Cp, o), k2) for o in o1s]
    in_specs += [pl.BlockSpec((8, o), k2) for o in o1s]
    out_specs = [pl.BlockSpec((1, S, Ks[i]), c0) for i in range(nscale)]
    out_shape = [jax.ShapeDtypeStruct((B, S, Ks[i]), jnp.int32)
                 for i in range(nscale)]
    out_specs += [pl.BlockSpec((1, S, N), c0) for _ in range(nscale)]
    out_shape += [jax.ShapeDtypeStruct((B, S, N), jnp.int32)
                  for _ in range(nscale)]
    if emit_g:
        out_specs += [pl.BlockSpec((1, N, o), c0) for o in o1s]
        out_shape += [jax.ShapeDtypeStruct((B, N, o), F32) for o in o1s]
    out_specs += [pl.BlockSpec((1, S, o), c0) for o in o1s]
    out_shape += [jax.ShapeDtypeStruct((B, S, o), F32) for o in o1s]
    args = [naug, xyz3, feats, lt] + (list(w1s) if emit_g else []) + list(wbs)
    return pl.pallas_call(
        functools.partial(_ball_body, r2s=tuple(r2s), Ks=tuple(Ks), N=N, S=S,
                          emit_g=emit_g),
        grid=(B,),
        in_specs=in_specs,
        out_specs=out_specs,
        out_shape=out_shape,
    )(*args)


def _mlp_body(*refs, K, has_w1):
    if has_w1:
        (g_ref, bias_ref, w1_ref, w2_ref, b2_ref, w3_ref, b3_ref,
         out_ref) = refs
        x = jnp.dot(g_ref[...], w1_ref[...],
                    preferred_element_type=F32)
    else:
        g_ref, bias_ref, w2_ref, b2_ref, w3_ref, b3_ref, out_ref = refs
        x = g_ref[...]
    BSK, O1 = x.shape
    BS = BSK // K
    x = x.reshape(BS, K, O1) + bias_ref[...][:, None, :]
    x = jnp.maximum(x, 0.0).reshape(BSK, O1)
    x = jnp.maximum(jnp.dot(x, w2_ref[...], preferred_element_type=F32)
                    + b2_ref[...], 0.0)
    x = jnp.maximum(jnp.dot(x, w3_ref[...], preferred_element_type=F32)
                    + b3_ref[...], 0.0)
    out_ref[...] = jnp.max(x.reshape(BS, K, -1), axis=1)


def _mlp(rows, bias, w1, w2, b2, w3, b3, K, BS):
    R, D = rows.shape
    M, O1 = bias.shape
    O2, O3 = w2.shape[1], w3.shape[1]
    i0 = lambda i: (i, 0)
    k2 = lambda i: (0, 0)
    in_specs = [pl.BlockSpec((BS * K, D), i0), pl.BlockSpec((BS, O1), i0)]
    args = [rows, bias]
    if w1 is not None:
        in_specs.append(pl.BlockSpec((D, O1), k2))
        args.append(w1)
    in_specs += [pl.BlockSpec((O1, O2), k2), pl.BlockSpec((1, O2), k2),
                 pl.BlockSpec((O2, O3), k2), pl.BlockSpec((1, O3), k2)]
    args += [w2, b2, w3, b3]
    return pl.pallas_call(
        functools.partial(_mlp_body, K=K, has_w1=w1 is not None),
        grid=(M // BS,),
        in_specs=in_specs,
        out_specs=pl.BlockSpec((BS, O3), i0),
        out_shape=jax.ShapeDtypeStruct((M, O3), F32),
    )(*args)


def _sa3_body(x_ref, w1, b1, w2, b2, w3, b3, out_ref, *, B, P):
    x = x_ref[...].reshape(B * P, -1)
    x = jnp.maximum(jnp.dot(x, w1[...], preferred_element_type=F32) + b1[...], 0.0)
    x = jnp.maximum(jnp.dot(x, w2[...], preferred_element_type=F32) + b2[...], 0.0)
    x = jnp.maximum(jnp.dot(x, w3[...], preferred_element_type=F32) + b3[...], 0.0)
    out_ref[...] = jnp.max(x.reshape(B, P, -1), axis=1)


def _sa3(feats, w1, b1, w2, b2, w3, b3):
    B, P, Cp = feats.shape
    O3 = w3.shape[1]
    return pl.pallas_call(
        functools.partial(_sa3_body, B=B, P=P),
        out_shape=jax.ShapeDtypeStruct((B, O3), F32),
    )(feats, w1, b1, w2, b2, w3, b3)


def _pad_rows(w, rows):
    return jnp.pad(w, ((0, rows - w.shape[0]), (0, 0)))


def _stage_weights(scale_params, c_pts, cp):
    out = []
    for layers in scale_params:
        w1t, b1 = _fold(layers[0])
        w2t, b2 = _fold(layers[1])
        w3t, b3 = _fold(layers[2])
        w1p = _pad_rows(w1t, cp)
        wb = jnp.concatenate([-w1t[c_pts:c_pts + 3], b1[None, :]], axis=0)
        wb = _pad_rows(wb, 8)
        out.append((w1p, wb, w2t, b2[None, :], w3t, b3[None, :]))
    return out


_SC_CORES, _SC_SUBCORES = 2, 16
_SC_WORKERS = _SC_CORES * _SC_SUBCORES
_CH = 128


def _sc_select_gather_body(rpw, N, K, S, cr, gr, ngr, nbuf, tgt_hbm, init_hbm,
                           table_hbm, out_hbm, idx_v, tgt_v, *rest):
    bufs = rest[:nbuf]
    ssems = rest[nbuf:2 * nbuf]
    gsem = rest[2 * nbuf]
    wid = lax.axis_index("s") * _SC_CORES + lax.axis_index("c")
    row0 = wid * rpw
    base_slot = row0 * K
    slab = rpw * K
    voff = (row0 // S) * N
    gpr = N // 16
    nch = rpw // cr
    ncpg = gr // _CH

    pltpu.sync_copy(init_hbm.at[pl.ds(base_slot, slab)], idx_v)
    lanes = lax.iota(jnp.int32, 16)

    def chunk(c, carry):
        pltpu.sync_copy(tgt_hbm.at[pl.ds((row0 + c * cr) * N, cr * N)], tgt_v)

        def group(i, carry2):
            t = lax.rem(i, gpr)
            tg = tgt_v[pl.ds(i * 16, 16)] - base_slot
            msk = tg < slab
            val = lanes + (t * 16 + voff)
            plsc.store_scatter(idx_v, (tg,), val, mask=msk)
            return carry2

        lax.fori_loop(0, cr * gpr, group, 0)
        return carry

    lax.fori_loop(0, nch, chunk, 0)

    def do_group(g, buf, ssem):
        cps = []
        for c in range(ncpg):
            cps.append(pltpu.async_copy(
                table_hbm.at[idx_v.at[pl.ds(g * gr + c * _CH, _CH)]],
                buf.at[pl.ds(c * _CH, _CH)], gsem))
        for cp in cps:
            cp.wait()
        pltpu.async_copy(buf, out_hbm.at[pl.ds(base_slot + g * gr, gr)], ssem)

    def drain_store(buf, ssem):
        pltpu.make_async_copy(out_hbm.at[pl.ds(0, gr)], buf, ssem).wait()

    nprime = min(nbuf, ngr)
    for g in range(nprime):
        do_group(g, bufs[g], ssems[g])
    if ngr > nbuf:

        def body(g, carry):
            b = lax.rem(g, nbuf)
            for k in range(nbuf):

                @pl.when(b == k)
                def _(k=k):
                    drain_store(bufs[k], ssems[k])
                    do_group(g, bufs[k], ssems[k])

            return carry

        lax.fori_loop(nbuf, ngr, body, 0)
    for g in range(nprime):
        drain_store(bufs[g], ssems[g])


def _sc_select_gather(tgt, init, table, N, K, S):
    M = tgt.shape[0]
    _, D = table.shape
    rpw = M // _SC_WORKERS
    slab = rpw * K
    gr = min(slab, max(_CH, 32768 // D))
    ngr = slab // gr
    nbuf = min(2, ngr)
    cr = min(max(1, (16 * 1024) // N), rpw)
    return pl.kernel(
        functools.partial(_sc_select_gather_body, rpw, N, K, S, cr, gr, ngr,
                          nbuf),
        mesh=plsc.VectorSubcoreMesh(core_axis_name="c", subcore_axis_name="s"),
        out_type=jax.ShapeDtypeStruct((M * K, D), F32),
        scratch_types=(
            [pltpu.VMEM((slab,), jnp.int32), pltpu.VMEM((cr * N,), jnp.int32)]
            + [pltpu.VMEM((gr, D), F32) for _ in range(nbuf)]
            + [pltpu.SemaphoreType.DMA for _ in range(nbuf + 1)]
        ),
        compiler_params=pltpu.CompilerParams(use_tc_tiling_on_sc=False,
                                             needs_layout_passes=False),
    )(tgt.reshape(-1), init.reshape(-1), table)


def _sa_msg(feats, naug, nxyz_b3s, weights, r2s, Ks, S, BS, raw_gather):
    B, N, Cp = feats.shape
    lt = jnp.triu(jnp.ones((N, N), F32))
    w1s = [w[0] for w in weights]
    wbs = [w[1] for w in weights]
    o1s = [w[0].shape[1] for w in weights]
    nscale = len(Ks)
    res = _ball(naug, nxyz_b3s, feats, lt, w1s, wbs, o1s, r2s, Ks,
                emit_g=not raw_gather)
    inits = res[:nscale]
    tgts = res[nscale:2 * nscale]
    if raw_gather:
        biases = res[2 * nscale:]
        gs = [None] * nscale
        table = feats.reshape(B * N, Cp)
    else:
        gs = res[2 * nscale:3 * nscale]
        biases = res[3 * nscale:]
        table = None
    rows = [_sc_select_gather(
        tgts[i].reshape(B * S, N), inits[i],
        table if raw_gather else gs[i].reshape(B * N, -1), N, Ks[i], S)
        for i in range(nscale)]
    outs = []
    for i in range(nscale):
        w1, _, w2, b2, w3, b3 = weights[i]
        o = _mlp(rows[i], biases[i].reshape(B * S, -1),
                 w1 if raw_gather else None, w2, b2, w3, b3, Ks[i], BS)
        outs.append(o.reshape(B, S, -1))
    return outs


def kernel(xyz, params):
    B, C6, N = xyz.shape
    coords = xyz[:, :3, :]
    norm = xyz[:, 3:, :]

    S1, Ks1 = 512, (16, 32, 128)
    r2s1 = tuple(np.float32(float(r) ** 2) for r in (0.1, 0.2, 0.4))
    fps_in = jnp.transpose(coords, (1, 0, 2))
    c1 = _fps(fps_in, S1)
    nxyz1 = jnp.transpose(c1, (1, 2, 0))
    naug1 = jnp.concatenate(
        [nxyz1, jnp.ones((B, S1, 1), F32), jnp.zeros((B, S1, 4), F32)], -1)
    feats1 = jnp.concatenate(
        [jnp.transpose(norm, (0, 2, 1)), jnp.transpose(coords, (0, 2, 1)),
         jnp.zeros((B, N, 10), F32)], -1)
    w_s1 = _stage_weights(params['sa1'], 3, 16)
    outs1 = _sa_msg(feats1, naug1, coords, w_s1, r2s1, Ks1, S1, BS=64,
                    raw_gather=True)
    l1_points = jnp.concatenate(outs1, -1)

    S2, Ks2 = 128, (32, 64, 128)
    r2s2 = tuple(np.float32(float(r) ** 2) for r in (0.2, 0.4, 0.8))
    c2 = _fps(c1, S2)
    nxyz2 = jnp.transpose(c2, (1, 2, 0))
    naug2 = jnp.concatenate(
        [nxyz2, jnp.ones((B, S2, 1), F32), jnp.zeros((B, S2, 4), F32)], -1)
    l1_xyz_b3s = jnp.transpose(c1, (1, 0, 2))
    feats2 = jnp.concatenate(
        [l1_points, nxyz1, jnp.zeros((B, S1, 384 - 323), F32)], -1)
    w_s2 = _stage_weights(params['sa2'], 320, 384)
    outs2 = _sa_msg(feats2, naug2, l1_xyz_b3s, w_s2, r2s2, Ks2, S2, BS=32,
                    raw_gather=False)
    l2_points = jnp.concatenate(outs2, -1)

    w1t, b1 = _fold(params['sa3'][0])
    w2t, b2 = _fold(params['sa3'][1])
    w3t, b3 = _fold(params['sa3'][2])
    cp3 = 768
    feats3 = jnp.concatenate(
        [nxyz2, l2_points, jnp.zeros((B, S2, cp3 - 643), F32)], -1)
    out = _sa3(feats3, _pad_rows(w1t, cp3), b1[None, :], w2t, b2[None, :],
               w3t, b3[None, :])
    return out


# speedup vs baseline: 1.0928x; 1.0444x over previous
import functools

import jax
import jax.numpy as jnp
import numpy as np
from jax import lax
from jax.experimental import pallas as pl
from jax.experimental.pallas import tpu as pltpu
from jax.experimental.pallas import tpu_sc as plsc

F32 = jnp.float32


def _fold(layer):
    s = layer['gamma'] / jnp.sqrt(layer['var'] + 1e-5)
    w = layer['W'] * s[:, None]
    b = (layer['b'] - layer['mean']) * s + layer['beta']
    return w.T, b


def _fps_body(xyz_ref, out_ref, *, npoint):
    xyz = xyz_ref[...]
    _, B, N = xyz.shape
    lane_n = jax.lax.broadcasted_iota(jnp.int32, (B, N), 1)
    lane_s = jax.lax.broadcasted_iota(jnp.int32, (3, B, npoint), 2)

    def body(i, st):
        dist, far, acc = st
        onehot = (lane_n == far).astype(F32)
        c = jnp.sum(xyz * onehot[None, :, :], axis=2, keepdims=True)
        acc = jnp.where(lane_s == i, c, acc)
        diff = xyz - c
        d = jnp.sum(diff * diff, axis=0)
        dist = jnp.minimum(dist, d)
        m = jnp.max(dist, axis=1, keepdims=True)
        far = jnp.min(jnp.where(dist == m, lane_n, N), axis=1, keepdims=True)
        return dist, far, acc

    dist0 = jnp.full((B, N), 1e10, F32)
    far0 = jnp.zeros((B, 1), jnp.int32)
    acc0 = jnp.zeros((3, B, npoint), F32)
    _, _, acc = jax.lax.fori_loop(0, npoint, body, (dist0, far0, acc0))
    out_ref[...] = acc


def _fps(xyz3bn, npoint):
    _, B, N = xyz3bn.shape
    return pl.pallas_call(
        functools.partial(_fps_body, npoint=npoint),
        out_shape=jax.ShapeDtypeStruct((3, B, npoint), F32),
    )(xyz3bn)


def _ball_body(naug_ref, xyz3_ref, feats_ref, lt_ref, *rest, r2s, Ks, N, S,
               emit_g):
    nscale = len(Ks)
    if emit_g:
        w1s = rest[:nscale]
        wbs = rest[nscale:2 * nscale]
        idx_refs = rest[2 * nscale:3 * nscale]
        tgt_refs = rest[3 * nscale:4 * nscale]
        g_refs = rest[4 * nscale:5 * nscale]
        bias_refs = rest[5 * nscale:6 * nscale]
    else:
        w1s = g_refs = None
        wbs = rest[:nscale]
        idx_refs = rest[nscale:2 * nscale]
        tgt_refs = rest[2 * nscale:3 * nscale]
        bias_refs = rest[3 * nscale:4 * nscale]

    b = pl.program_id(0)
    naug = naug_ref[0]
    x3 = xyz3_ref[0]
    lt = lt_ref[...]

    nxyz = naug[:, :3]
    sc = jnp.sum(nxyz * nxyz, axis=1, keepdims=True)
    sx = jnp.sum(x3 * x3, axis=0, keepdims=True)
    cross = jnp.dot(nxyz, x3, preferred_element_type=F32)
    sq = (sc + sx) - 2.0 * cross

    lane_j = jax.lax.broadcasted_iota(jnp.int32, (S, N), 1)
    row_s = jax.lax.broadcasted_iota(jnp.int32, (S, N), 0)
    M = pl.num_programs(0) * S
    for i in range(nscale):
        K = Ks[i]
        validb = sq <= r2s[i]
        valid = validb.astype(F32)
        rc = jnp.dot(valid, lt, preferred_element_type=F32)
        inr = jnp.logical_and(validb, rc <= float(K))
        tgt = jnp.where(inr,
                        (b * S + row_s) * K + (rc.astype(jnp.int32) - 1),
                        M * K)
        tgt_refs[i][0] = tgt
        first = jnp.min(jnp.where(validb, lane_j, N), axis=1, keepdims=True)
        idx_refs[i][0] = jnp.broadcast_to(first + b * N, (S, K))
        if emit_g:
            g_refs[i][0] = jnp.dot(feats_ref[0], w1s[i][...],
                                   preferred_element_type=F32)
        bias_refs[i][0] = jnp.dot(naug, wbs[i][...],
                                  preferred_element_type=F32)


def _ball(naug, xyz3, feats, lt, w1s, wbs, o1s, r2s, Ks, emit_g):
    B, S, _ = naug.shape
    N = xyz3.shape[2]
    Cp = feats.shape[2]
    nscale = len(Ks)
    c0 = lambda i: (i, 0, 0)
    k2 = lambda i: (0, 0)
    in_specs = [
        pl.BlockSpec((1, S, 8), c0),
        pl.BlockSpec((1, 3, N), c0),
        pl.BlockSpec((1, N, Cp), c0),
        pl.BlockSpec((N, N), k2),
    ]
    if emit_g:
        in_specs += [pl.BlockSpec((Cp, o), k2) for o in o1s]
    in_specs += [pl.BlockSpec((8, o), k2) for o in o1s]
    out_specs = [pl.BlockSpec((1, S, Ks[i]), c0) for i in range(nscale)]
    out_shape = [jax.ShapeDtypeStruct((B, S, Ks[i]), jnp.int32)
                 for i in range(nscale)]
    out_specs += [pl.BlockSpec((1, S, N), c0) for _ in range(nscale)]
    out_shape += [jax.ShapeDtypeStruct((B, S, N), jnp.int32)
                  for _ in range(nscale)]
    if emit_g:
        out_specs += [pl.BlockSpec((1, N, o), c0) for o in o1s]
        out_shape += [jax.ShapeDtypeStruct((B, N, o), F32) for o in o1s]
    out_specs += [pl.BlockSpec((1, S, o), c0) for o in o1s]
    out_shape += [jax.ShapeDtypeStruct((B, S, o), F32) for o in o1s]
    args = [naug, xyz3, feats, lt] + (list(w1s) if emit_g else []) + list(wbs)
    return pl.pallas_call(
        functools.partial(_ball_body, r2s=tuple(r2s), Ks=tuple(Ks), N=N, S=S,
                          emit_g=emit_g),
        grid=(B,),
        in_specs=in_specs,
        out_specs=out_specs,
        out_shape=out_shape,
    )(*args)


def _mlp_body(*refs, K, has_w1):
    if has_w1:
        (g_ref, bias_ref, w1_ref, w2_ref, b2_ref, w3_ref, b3_ref,
         out_ref) = refs
        x = jnp.dot(g_ref[...], w1_ref[...],
                    preferred_element_type=F32)
    else:
        g_ref, bias_ref, w2_ref, b2_ref, w3_ref, b3_ref, out_ref = refs
        x = g_ref[...]
    BSK, O1 = x.shape
    BS = BSK // K
    x = x.reshape(BS, K, O1) + bias_ref[...][:, None, :]
    x = jnp.maximum(x, 0.0).reshape(BSK, O1)
    x = jnp.maximum(jnp.dot(x, w2_ref[...], preferred_element_type=F32)
                    + b2_ref[...], 0.0)
    x = jnp.maximum(jnp.dot(x, w3_ref[...], preferred_element_type=F32)
                    + b3_ref[...], 0.0)
    out_ref[...] = jnp.max(x.reshape(BS, K, -1), axis=1)


def _mlp(rows, bias, w1, w2, b2, w3, b3, K, BS):
    R, D = rows.shape
    M, O1 = bias.shape
    O2, O3 = w2.shape[1], w3.shape[1]
    i0 = lambda i: (i, 0)
    k2 = lambda i: (0, 0)
    in_specs = [pl.BlockSpec((BS * K, D), i0), pl.BlockSpec((BS, O1), i0)]
    args = [rows, bias]
    if w1 is not None:
        in_specs.append(pl.BlockSpec((D, O1), k2))
        args.append(w1)
    in_specs += [pl.BlockSpec((O1, O2), k2), pl.BlockSpec((1, O2), k2),
                 pl.BlockSpec((O2, O3), k2), pl.BlockSpec((1, O3), k2)]
    args += [w2, b2, w3, b3]
    return pl.pallas_call(
        functools.partial(_mlp_body, K=K, has_w1=w1 is not None),
        grid=(M // BS,),
        in_specs=in_specs,
        out_specs=pl.BlockSpec((BS, O3), i0),
        out_shape=jax.ShapeDtypeStruct((M, O3), F32),
    )(*args)


def _sa3_body(x_ref, w1, b1, w2, b2, w3, b3, out_ref, *, B, P):
    x = x_ref[...].reshape(B * P, -1)
    x = jnp.maximum(jnp.dot(x, w1[...], preferred_element_type=F32) + b1[...], 0.0)
    x = jnp.maximum(jnp.dot(x, w2[...], preferred_element_type=F32) + b2[...], 0.0)
    x = jnp.maximum(jnp.dot(x, w3[...], preferred_element_type=F32) + b3[...], 0.0)
    out_ref[...] = jnp.max(x.reshape(B, P, -1), axis=1)


def _sa3(feats, w1, b1, w2, b2, w3, b3):
    B, P, Cp = feats.shape
    O3 = w3.shape[1]
    return pl.pallas_call(
        functools.partial(_sa3_body, B=B, P=P),
        out_shape=jax.ShapeDtypeStruct((B, O3), F32),
    )(feats, w1, b1, w2, b2, w3, b3)


def _pad_rows(w, rows):
    return jnp.pad(w, ((0, rows - w.shape[0]), (0, 0)))


def _stage_weights(scale_params, c_pts, cp):
    out = []
    for layers in scale_params:
        w1t, b1 = _fold(layers[0])
        w2t, b2 = _fold(layers[1])
        w3t, b3 = _fold(layers[2])
        w1p = _pad_rows(w1t, cp)
        wb = jnp.concatenate([-w1t[c_pts:c_pts + 3], b1[None, :]], axis=0)
        wb = _pad_rows(wb, 8)
        out.append((w1p, wb, w2t, b2[None, :], w3t, b3[None, :]))
    return out


_SC_CORES, _SC_SUBCORES = 2, 16
_SC_WORKERS = _SC_CORES * _SC_SUBCORES
_CH = 128


def _sc_select_gather_body(rpw, N, K, S, cr, gr, ngr, nbuf, tgt_hbm, init_hbm,
                           table_hbm, out_hbm, idx_v, tgt_v, *rest):
    bufs = rest[:nbuf]
    ssems = rest[nbuf:2 * nbuf]
    gsem = rest[2 * nbuf]
    wid = lax.axis_index("s") * _SC_CORES + lax.axis_index("c")
    row0 = wid * rpw
    base_slot = row0 * K
    slab = rpw * K
    voff = (row0 // S) * N
    gpr = N // 16
    nch = rpw // cr
    ncpg = gr // _CH

    pltpu.sync_copy(init_hbm.at[pl.ds(base_slot, slab)], idx_v)
    lanes = lax.iota(jnp.int32, 16)

    def chunk(c, carry):
        pltpu.sync_copy(tgt_hbm.at[pl.ds((row0 + c * cr) * N, cr * N)], tgt_v)

        def group(i, carry2):
            t = lax.rem(i, gpr)
            tg = tgt_v[pl.ds(i * 16, 16)] - base_slot
            msk = tg < slab
            val = lanes + (t * 16 + voff)
            plsc.store_scatter(idx_v, (tg,), val, mask=msk)
            return carry2

        lax.fori_loop(0, cr * gpr, group, 0)
        return carry

    lax.fori_loop(0, nch, chunk, 0)

    def do_group(g, buf, ssem):
        cps = []
        for c in range(ncpg):
            cps.append(pltpu.async_copy(
                table_hbm.at[idx_v.at[pl.ds(g * gr + c * _CH, _CH)]],
                buf.at[pl.ds(c * _CH, _CH)], gsem))
        for cp in cps:
            cp.wait()
        pltpu.async_copy(buf, out_hbm.at[pl.ds(base_slot + g * gr, gr)], ssem)

    def drain_store(buf, ssem):
        pltpu.make_async_copy(out_hbm.at[pl.ds(0, gr)], buf, ssem).wait()

    nprime = min(nbuf, ngr)
    for g in range(nprime):
        do_group(g, bufs[g], ssems[g])
    if ngr > nbuf:

        def body(g, carry):
            b = lax.rem(g, nbuf)
            for k in range(nbuf):

                @pl.when(b == k)
                def _(k=k):
                    drain_store(bufs[k], ssems[k])
                    do_group(g, bufs[k], ssems[k])

            return carry

        lax.fori_loop(nbuf, ngr, body, 0)
    for g in range(nprime):
        drain_store(bufs[g], ssems[g])


def _sc_select_gather(tgt, init, table, N, K, S):
    M = tgt.shape[0]
    _, D = table.shape
    rpw = M // _SC_WORKERS
    slab = rpw * K
    gr = min(slab, max(_CH, 32768 // D))
    ngr = slab // gr
    nbuf = min(2, ngr)
    cr = min(max(1, (16 * 1024) // N), rpw)
    return pl.kernel(
        functools.partial(_sc_select_gather_body, rpw, N, K, S, cr, gr, ngr,
                          nbuf),
        mesh=plsc.VectorSubcoreMesh(core_axis_name="c", subcore_axis_name="s"),
        out_type=jax.ShapeDtypeStruct((M * K, D), F32),
        scratch_types=(
            [pltpu.VMEM((slab,), jnp.int32), pltpu.VMEM((cr * N,), jnp.int32)]
            + [pltpu.VMEM((gr, D), F32) for _ in range(nbuf)]
            + [pltpu.SemaphoreType.DMA for _ in range(nbuf + 1)]
        ),
        compiler_params=pltpu.CompilerParams(use_tc_tiling_on_sc=False,
                                             needs_layout_passes=False),
    )(tgt.reshape(-1), init.reshape(-1), table)


def _sa_msg(feats, naug, nxyz_b3s, weights, r2s, Ks, S, BS, raw_gather):
    B, N, Cp = feats.shape
    lt = jnp.triu(jnp.ones((N, N), F32))
    w1s = [w[0] for w in weights]
    wbs = [w[1] for w in weights]
    o1s = [w[0].shape[1] for w in weights]
    nscale = len(Ks)
    res = _ball(naug, nxyz_b3s, feats, lt, w1s, wbs, o1s, r2s, Ks,
                emit_g=not raw_gather)
    inits = res[:nscale]
    tgts = res[nscale:2 * nscale]
    if raw_gather:
        biases = res[2 * nscale:]
        gs = [None] * nscale
        table = feats.reshape(B * N, Cp)
    else:
        gs = res[2 * nscale:3 * nscale]
        biases = res[3 * nscale:]
        table = None
    rows = [_sc_select_gather(
        tgts[i].reshape(B * S, N), inits[i],
        table if raw_gather else gs[i].reshape(B * N, -1), N, Ks[i], S)
        for i in range(nscale)]
    outs = []
    for i in range(nscale):
        w1, _, w2, b2, w3, b3 = weights[i]
        o = _mlp(rows[i], biases[i].reshape(B * S, -1),
                 w1 if raw_gather else None, w2, b2, w3, b3, Ks[i], BS)
        outs.append(o.reshape(B, S, -1))
    return outs


def kernel(xyz, params):
    B, C6, N = xyz.shape
    coords = xyz[:, :3, :]
    norm = xyz[:, 3:, :]

    S1, Ks1 = 512, (16, 32, 128)
    r2s1 = tuple(np.float32(float(r) ** 2) for r in (0.1, 0.2, 0.4))
    fps_in = jnp.transpose(coords, (1, 0, 2))
    c1 = _fps(fps_in, S1)
    nxyz1 = jnp.transpose(c1, (1, 2, 0))
    naug1 = jnp.concatenate(
        [nxyz1, jnp.ones((B, S1, 1), F32), jnp.zeros((B, S1, 4), F32)], -1)
    feats1 = jnp.concatenate(
        [jnp.transpose(norm, (0, 2, 1)), jnp.transpose(coords, (0, 2, 1)),
         jnp.zeros((B, N, 10), F32)], -1)
    w_s1 = _stage_weights(params['sa1'], 3, 16)
    outs1 = _sa_msg(feats1, naug1, coords, w_s1, r2s1, Ks1, S1, BS=128,
                    raw_gather=True)
    l1_points = jnp.concatenate(outs1, -1)

    S2, Ks2 = 128, (32, 64, 128)
    r2s2 = tuple(np.float32(float(r) ** 2) for r in (0.2, 0.4, 0.8))
    c2 = _fps(c1, S2)
    nxyz2 = jnp.transpose(c2, (1, 2, 0))
    naug2 = jnp.concatenate(
        [nxyz2, jnp.ones((B, S2, 1), F32), jnp.zeros((B, S2, 4), F32)], -1)
    l1_xyz_b3s = jnp.transpose(c1, (1, 0, 2))
    feats2 = jnp.concatenate(
        [l1_points, nxyz1, jnp.zeros((B, S1, 384 - 323), F32)], -1)
    w_s2 = _stage_weights(params['sa2'], 320, 384)
    outs2 = _sa_msg(feats2, naug2, l1_xyz_b3s, w_s2, r2s2, Ks2, S2, BS=64,
                    raw_gather=False)
    l2_points = jnp.concatenate(outs2, -1)

    w1t, b1 = _fold(params['sa3'][0])
    w2t, b2 = _fold(params['sa3'][1])
    w3t, b3 = _fold(params['sa3'][2])
    cp3 = 768
    feats3 = jnp.concatenate(
        [nxyz2, l2_points, jnp.zeros((B, S2, cp3 - 643), F32)], -1)
    out = _sa3(feats3, _pad_rows(w1t, cp3), b1[None, :], w2t, b2[None, :],
               w3t, b3[None, :])
    return out
